# baseline (speedup 1.0000x reference)
import functools

import jax
import jax.numpy as jnp
from jax import lax
from jax.experimental import pallas as pl
from jax.experimental.pallas import tpu as pltpu
from jax.experimental.pallas import tpu_sc as plsc

S = 2048
B = 4
D = 1024
BD = B * D

NC = 2
NS = 16
NW = NC * NS
RPW = S // NW
CH = 8
NCH = RPW // CH
LANES = 16


def _sc_body(x_hbm, pe_hbm, out_hbm, xbuf, pbuf, in_sem, out_sem):
    wid = lax.axis_index("s") * NC + lax.axis_index("c")
    base = wid * RPW

    def compute_chunk():
        def rbody(r, carry):
            def jbody(j, carry2):
                col = j * LANES
                pe_v = pbuf[r, pl.ds(col, LANES)]
                for b in range(B):
                    off = b * D + col
                    xbuf[r, pl.ds(off, LANES)] = xbuf[r, pl.ds(off, LANES)] + pe_v
                return carry2
            return lax.fori_loop(0, D // LANES, jbody, carry, unroll=4)
        lax.fori_loop(0, CH, rbody, 0)

    for c in range(NCH):
        row0 = base + c * CH
        pltpu.sync_copy(x_hbm.at[pl.ds(row0, CH)], xbuf)
        pltpu.sync_copy(pe_hbm.at[pl.ds(row0, CH)], pbuf)
        compute_chunk()
        pltpu.sync_copy(xbuf, out_hbm.at[pl.ds(row0, CH)])


@functools.partial(
    pl.kernel,
    mesh=plsc.VectorSubcoreMesh(core_axis_name="c", subcore_axis_name="s"),
    out_type=jax.ShapeDtypeStruct((S, BD), jnp.float32),
    scratch_types=[
        pltpu.VMEM((CH, BD), jnp.float32),
        pltpu.VMEM((CH, D), jnp.float32),
        pltpu.SemaphoreType.DMA,
        pltpu.SemaphoreType.DMA,
    ],
)
def _pe_add_sc(x_hbm, pe_hbm, out_hbm, xbuf, pbuf, in_sem, out_sem):
    _sc_body(x_hbm, pe_hbm, out_hbm, xbuf, pbuf, in_sem, out_sem)


def kernel(x, pos_embed):
    out = _pe_add_sc(x.reshape(S, BD), pos_embed)
    return out.reshape(S, B, D)


# speedup vs baseline: 1.6530x; 1.6530x over previous
import functools

import jax
import jax.numpy as jnp
from jax import lax
from jax.experimental import pallas as pl
from jax.experimental.pallas import tpu as pltpu
from jax.experimental.pallas import tpu_sc as plsc

S = 2048
B = 4
D = 1024
BD = B * D

NC = 2
NS = 16
NW = NC * NS
RPW = S // NW
CH = 8
NCH = RPW // CH
NBUF = 3
LANES = 16


def _compute_chunk(xb, pb):
    def rbody(r, carry):
        for j in range(D // LANES):
            col = j * LANES
            pe_v = pb[r, pl.ds(col, LANES)]
            for b in range(B):
                off = b * D + col
                xb[r, pl.ds(off, LANES)] = xb[r, pl.ds(off, LANES)] + pe_v
        return carry
    lax.fori_loop(0, CH, rbody, 0)


def _sc_body(x_hbm, pe_hbm, out_hbm, bufs):
    wid = lax.axis_index("s") * NC + lax.axis_index("c")
    base = wid * RPW

    def start_in(c):
        xb, pb, isem, _ = bufs[c % NBUF]
        row0 = base + c * CH
        cx = pltpu.async_copy(x_hbm.at[pl.ds(row0, CH)], xb, isem)
        cp = pltpu.async_copy(pe_hbm.at[pl.ds(row0, CH)], pb, isem)
        return cx, cp

    in_fl = {0: start_in(0), 1: start_in(1)}
    out_fl = {}
    for c in range(NCH):
        xb, pb, _, osem = bufs[c % NBUF]
        cx, cp = in_fl.pop(c)
        cx.wait()
        cp.wait()
        _compute_chunk(xb, pb)
        out_fl[c] = pltpu.async_copy(
            xb, out_hbm.at[pl.ds(base + c * CH, CH)], osem)
        nxt = c + 2
        if nxt < NCH:
            prev = nxt - NBUF
            if prev >= 0:
                out_fl.pop(prev).wait()
            in_fl[nxt] = start_in(nxt)
    for o in out_fl.values():
        o.wait()


@functools.partial(
    pl.kernel,
    mesh=plsc.VectorSubcoreMesh(core_axis_name="c", subcore_axis_name="s"),
    out_type=jax.ShapeDtypeStruct((S, BD), jnp.float32),
    scratch_types=(
        [pltpu.VMEM((CH, BD), jnp.float32) for _ in range(NBUF)]
        + [pltpu.VMEM((CH, D), jnp.float32) for _ in range(NBUF)]
        + [pltpu.SemaphoreType.DMA for _ in range(2 * NBUF)]
    ),
)
def _pe_add_sc(x_hbm, pe_hbm, out_hbm,
               xb0, xb1, xb2, pb0, pb1, pb2,
               is0, is1, is2, os0, os1, os2):
    bufs = ((xb0, pb0, is0, os0), (xb1, pb1, is1, os1), (xb2, pb2, is2, os2))
    _sc_body(x_hbm, pe_hbm, out_hbm, bufs)


def kernel(x, pos_embed):
    out = _pe_add_sc(x.reshape(S, BD), pos_embed)
    return out.reshape(S, B, D)


# speedup vs baseline: 3.2729x; 1.9801x over previous
import functools

import jax
import jax.numpy as jnp
from jax import lax
from jax.experimental import pallas as pl
from jax.experimental.pallas import tpu as pltpu
from jax.experimental.pallas import tpu_sc as plsc

S = 2048
B = 4
D = 1024
BD = B * D

NC = 2
NS = 16
NW = NC * NS
RPW = S // NW
CH = 8
NCH = RPW // CH
NBUF = 3
LANES = 16


def _compute_chunk(xb, pb):
    def rbody(r, carry):
        for j in range(D // LANES):
            col = j * LANES
            pe_v = pb[r, pl.ds(col, LANES)]
            for b in range(B):
                xb[r, b, pl.ds(col, LANES)] = xb[r, b, pl.ds(col, LANES)] + pe_v
        return carry
    lax.fori_loop(0, CH, rbody, 0)


def _sc_body(x_hbm, pe_hbm, out_hbm, bufs):
    wid = lax.axis_index("s") * NC + lax.axis_index("c")
    base = wid * RPW

    def start_in(c):
        xb, pb, isem, _ = bufs[c % NBUF]
        row0 = base + c * CH
        cx = pltpu.async_copy(x_hbm.at[pl.ds(row0, CH)], xb, isem)
        cp = pltpu.async_copy(pe_hbm.at[pl.ds(row0, CH)], pb, isem)
        return cx, cp

    in_fl = {0: start_in(0), 1: start_in(1)}
    out_fl = {}
    for c in range(NCH):
        xb, pb, _, osem = bufs[c % NBUF]
        cx, cp = in_fl.pop(c)
        cx.wait()
        cp.wait()
        _compute_chunk(xb, pb)
        out_fl[c] = pltpu.async_copy(
            xb, out_hbm.at[pl.ds(base + c * CH, CH)], osem)
        nxt = c + 2
        if nxt < NCH:
            prev = nxt - NBUF
            if prev >= 0:
                out_fl.pop(prev).wait()
            in_fl[nxt] = start_in(nxt)
    for o in out_fl.values():
        o.wait()


@functools.partial(
    pl.kernel,
    mesh=plsc.VectorSubcoreMesh(core_axis_name="c", subcore_axis_name="s"),
    out_type=jax.ShapeDtypeStruct((S, B, D), jnp.float32),
    scratch_types=(
        [pltpu.VMEM((CH, B, D), jnp.float32) for _ in range(NBUF)]
        + [pltpu.VMEM((CH, D), jnp.float32) for _ in range(NBUF)]
        + [pltpu.SemaphoreType.DMA for _ in range(2 * NBUF)]
    ),
)
def _pe_add_sc(x_hbm, pe_hbm, out_hbm,
               xb0, xb1, xb2, pb0, pb1, pb2,
               is0, is1, is2, os0, os1, os2):
    bufs = ((xb0, pb0, is0, os0), (xb1, pb1, is1, os1), (xb2, pb2, is2, os2))
    _sc_body(x_hbm, pe_hbm, out_hbm, bufs)


def kernel(x, pos_embed):
    return _pe_add_sc(x, pos_embed)
